# baseline (speedup 1.0000x reference)
import functools

import jax
import jax.numpy as jnp
from jax import lax
from jax.experimental import pallas as pl
from jax.experimental.pallas import tpu as pltpu
from jax.experimental.pallas import tpu_sc as plsc

NUM_FIELDS = 26
FIELD_DIM = 40000
EMBED_DIM = 16
BATCH = 16384

NC, NS = 2, 16
NW = NC * NS
TOTAL = BATCH * NUM_FIELDS
PER_W = TOTAL // NW
NCH = 4
CH = PER_W // NCH


def _gather_body(table_hbm, idx_hbm, out_hbm,
                 idx0, idx1, rows0, rows1, sem0, sem1):
    wid = lax.axis_index("s") * NC + lax.axis_index("c")
    base = wid * PER_W
    idx_b = (idx0, idx1)
    rows_b = (rows0, rows1)
    sems = (sem0, sem1)

    pltpu.sync_copy(idx_hbm.at[pl.ds(base, CH)], idx_b[0])
    cps = [None, None]
    cps[0] = pltpu.async_copy(table_hbm.at[idx_b[0]], rows_b[0], sems[0])
    for s in range(1, NCH):
        b = s % 2
        pltpu.sync_copy(idx_hbm.at[pl.ds(base + s * CH, CH)], idx_b[b])
        cps[b] = pltpu.async_copy(table_hbm.at[idx_b[b]], rows_b[b], sems[b])
        pb = (s - 1) % 2
        cps[pb].wait()
        pltpu.sync_copy(rows_b[pb], out_hbm.at[pl.ds(base + (s - 1) * CH, CH)])
    lb = (NCH - 1) % 2
    cps[lb].wait()
    pltpu.sync_copy(rows_b[lb], out_hbm.at[pl.ds(base + (NCH - 1) * CH, CH)])


@jax.jit
def kernel(x, weight):
    offsets = jnp.arange(NUM_FIELDS, dtype=x.dtype) * FIELD_DIM
    idx = (x + offsets[None, :]).reshape(TOTAL)
    mesh = plsc.VectorSubcoreMesh(core_axis_name="c", subcore_axis_name="s")
    rows = pl.kernel(
        _gather_body,
        out_type=jax.ShapeDtypeStruct((TOTAL, EMBED_DIM), jnp.float32),
        mesh=mesh,
        scratch_types=[
            pltpu.VMEM((CH,), jnp.int32),
            pltpu.VMEM((CH,), jnp.int32),
            pltpu.VMEM((CH, EMBED_DIM), jnp.float32),
            pltpu.VMEM((CH, EMBED_DIM), jnp.float32),
            pltpu.SemaphoreType.DMA,
            pltpu.SemaphoreType.DMA,
        ],
        compiler_params=pltpu.CompilerParams(use_tc_tiling_on_sc=False),
    )(weight, idx)
    return rows.reshape(BATCH, NUM_FIELDS, EMBED_DIM)
